# baseline (speedup 1.0000x reference)
import jax
import jax.numpy as jnp
from jax.experimental import pallas as pl
from jax.experimental.pallas import tpu as pltpu

_N = 1024
_BJ = 128
_NBLK = _N // _BJ


def _acc_kernel(z_ref, y_ref, out_ref, xn_ref):
    b = pl.program_id(0)
    x = z_ref[:]
    yb = y_ref[:]

    @pl.when(b == 0)
    def _init():
        xn_ref[...] = jnp.sqrt(jnp.sum(x * x, axis=1, keepdims=True))
        out_ref[...] = jnp.zeros_like(out_ref)

    num = jax.lax.dot_general(
        x, yb,
        dimension_numbers=(((1,), (1,)), ((), ())),
        preferred_element_type=jnp.float32,
    )
    xn = xn_ref[...]
    yn = jnp.sqrt(jnp.sum(yb * yb, axis=1))
    denom = jnp.maximum(xn * yn[None, :], 1e-8)
    simt = num / denom
    row = jax.lax.broadcasted_iota(jnp.int32, (_N, _BJ), 0)
    col = jax.lax.broadcasted_iota(jnp.int32, (_N, _BJ), 1) + b * _BJ
    d = jnp.sum(jnp.where(row == col, simt, 0.0), axis=0, keepdims=True)
    beats = (simt > d) | ((simt == d) & (row < col))
    cnt = jnp.sum(jnp.where(beats, 1.0, 0.0), axis=0, keepdims=True)
    top1 = jnp.sum(jnp.where(cnt == 0.0, 1.0, 0.0), axis=1, keepdims=True)
    top10 = jnp.sum(jnp.where(cnt < 10.0, 1.0, 0.0), axis=1, keepdims=True)
    out_ref[...] += jnp.concatenate([top1, top10], axis=1) * (1.0 / _N)


def kernel(Z, Y):
    out = pl.pallas_call(
        _acc_kernel,
        grid=(_NBLK,),
        in_specs=[
            pl.BlockSpec((_N, _N), lambda b: (0, 0)),
            pl.BlockSpec((_BJ, _N), lambda b: (b, 0)),
        ],
        out_specs=pl.BlockSpec((1, 2), lambda b: (0, 0)),
        out_shape=jax.ShapeDtypeStruct((1, 2), jnp.float32),
        scratch_shapes=[pltpu.VMEM((_N, 1), jnp.float32)],
    )(Z, Y)
    return (out[0, 0], out[0, 1])


# speedup vs baseline: 1.4849x; 1.4849x over previous
import jax
import jax.numpy as jnp
from jax.experimental import pallas as pl
from jax.experimental.pallas import tpu as pltpu

_N = 1024
_BJ = 512
_NBLK = _N // _BJ


def _acc_kernel(z_ref, y_ref, out_ref, xn_ref):
    b = pl.program_id(0)
    x = z_ref[:]
    yb = y_ref[:]

    @pl.when(b == 0)
    def _init():
        xn_ref[...] = jnp.sqrt(jnp.sum(x * x, axis=1, keepdims=True))
        out_ref[...] = jnp.zeros_like(out_ref)

    num = jax.lax.dot_general(
        x, yb,
        dimension_numbers=(((1,), (1,)), ((), ())),
        preferred_element_type=jnp.float32,
    )
    xn = xn_ref[...]
    yn = jnp.sqrt(jnp.sum(yb * yb, axis=1))
    denom = jnp.maximum(xn * yn[None, :], 1e-8)
    simt = num / denom
    row = jax.lax.broadcasted_iota(jnp.int32, (_N, _BJ), 0)
    col = jax.lax.broadcasted_iota(jnp.int32, (_N, _BJ), 1) + b * _BJ
    d = jnp.sum(jnp.where(row == col, simt, 0.0), axis=0, keepdims=True)
    beats = (simt > d) | ((simt == d) & (row < col))
    cnt = jnp.sum(jnp.where(beats, 1.0, 0.0), axis=0, keepdims=True)
    top1 = jnp.sum(jnp.where(cnt == 0.0, 1.0, 0.0), axis=1, keepdims=True)
    top10 = jnp.sum(jnp.where(cnt < 10.0, 1.0, 0.0), axis=1, keepdims=True)
    out_ref[...] += jnp.concatenate([top1, top10], axis=1) * (1.0 / _N)


def kernel(Z, Y):
    out = pl.pallas_call(
        _acc_kernel,
        grid=(_NBLK,),
        in_specs=[
            pl.BlockSpec((_N, _N), lambda b: (0, 0)),
            pl.BlockSpec((_BJ, _N), lambda b: (b, 0)),
        ],
        out_specs=pl.BlockSpec((1, 2), lambda b: (0, 0)),
        out_shape=jax.ShapeDtypeStruct((1, 2), jnp.float32),
        scratch_shapes=[pltpu.VMEM((_N, 1), jnp.float32)],
    )(Z, Y)
    return (out[0, 0], out[0, 1])
